# baseline (speedup 1.0000x reference)
import functools

import jax
import jax.numpy as jnp
from jax import lax
from jax.experimental import pallas as pl
from jax.experimental.pallas import tpu as pltpu
from jax.experimental.pallas import tpu_sc as plsc

_B = 16384
_D = 64
_NC = 2
_NS = 16
_NW = _NC * _NS
_BPW = _B // _NW
_WAVE = 32
_NWAVE = _BPW // _WAVE


def _body(users_hbm, items_hbm, ut3_hbm, it3_hbm, out_hbm,
          uidx_v, iidx_v, ublk_v, iblk_v, out_v, tile_v, sem_u, sem_i):
    cid = lax.axis_index("c")
    sid = lax.axis_index("s")
    wid = sid * _NC + cid
    base = wid * _BPW

    pltpu.sync_copy(users_hbm.at[pl.ds(base, _BPW)], uidx_v)
    pltpu.sync_copy(items_hbm.at[pl.ds(base, _BPW)], iidx_v)

    lanes = lax.iota(jnp.int32, 16)

    def wave_body(w, carry):
        e0 = w * _WAVE
        for g in range(_WAVE // 16):
            ut_vec = lax.shift_right_logical(
                uidx_v[pl.ds(e0 + g * 16, 16)], 3)
            it_vec = lax.shift_right_logical(
                iidx_v[pl.ds(e0 + g * 16, 16)], 3)
            for i in range(16):
                j = g * 16 + i
                pltpu.async_copy(ut3_hbm.at[ut_vec[i]], ublk_v.at[j], sem_u)
                pltpu.async_copy(it3_hbm.at[it_vec[i]], iblk_v.at[j], sem_i)
        pltpu.make_async_copy(ut3_hbm.at[pl.ds(0, _WAVE)], ublk_v,
                              sem_u).wait()
        pltpu.make_async_copy(it3_hbm.at[pl.ds(0, _WAVE)], iblk_v,
                              sem_i).wait()

        for g in range(_WAVE // 16):
            us_vec = jnp.bitwise_and(uidx_v[pl.ds(e0 + g * 16, 16)], 7)
            is_vec = jnp.bitwise_and(iidx_v[pl.ds(e0 + g * 16, 16)], 7)
            for i in range(16):
                j = g * 16 + i
                su = us_vec[i]
                si = is_vec[i]
                acc = ublk_v[j, su, pl.ds(0, 16)] * \
                    iblk_v[j, si, pl.ds(0, 16)]
                for c in range(1, 4):
                    acc = acc + ublk_v[j, su, pl.ds(c * 16, 16)] * \
                        iblk_v[j, si, pl.ds(c * 16, 16)]
                plsc.store_scatter(tile_v, [lanes * 16 + i], acc)
            rowsum = tile_v[pl.ds(0, 16)]
            for r in range(1, 16):
                rowsum = rowsum + tile_v[pl.ds(r * 16, 16)]
            out_v[pl.ds(e0 + g * 16, 16)] = rowsum
        return carry

    lax.fori_loop(0, _NWAVE, wave_body, 0)

    pltpu.sync_copy(out_v, out_hbm.at[pl.ds(base, _BPW)])


def _tc_transpose(tT):
    n = tT.shape[1]
    blk = 512
    grid = (n + blk - 1) // blk

    def body(i_ref, o_ref):
        o_ref[...] = i_ref[...].T

    return pl.pallas_call(
        body,
        grid=(grid,),
        in_specs=[pl.BlockSpec((_D, blk), lambda j: (0, j))],
        out_specs=pl.BlockSpec((blk, _D), lambda j: (j, 0)),
        out_shape=jax.ShapeDtypeStruct((n, _D), jnp.float32),
    )(tT)


@jax.jit
def _run(users, items, user_table, item_table):
    mesh = plsc.VectorSubcoreMesh(core_axis_name="c", subcore_axis_name="s")
    f = pl.kernel(
        _body,
        mesh=mesh,
        out_type=jax.ShapeDtypeStruct((_B,), jnp.float32),
        scratch_types=[
            pltpu.VMEM((_BPW,), jnp.int32),
            pltpu.VMEM((_BPW,), jnp.int32),
            pltpu.VMEM((_WAVE, 8, _D), jnp.float32),
            pltpu.VMEM((_WAVE, 8, _D), jnp.float32),
            pltpu.VMEM((_BPW,), jnp.float32),
            pltpu.VMEM((256,), jnp.float32),
            pltpu.SemaphoreType.DMA,
            pltpu.SemaphoreType.DMA,
        ],
        compiler_params=pltpu.CompilerParams(needs_layout_passes=False),
    )
    ut3 = _tc_transpose(user_table.T).reshape(125000, 8, _D)
    it3 = item_table.reshape(125000, 8, _D)
    return f(users, items, ut3, it3)


def kernel(users, items, user_table, item_table):
    return _run(users.astype(jnp.int32), items.astype(jnp.int32),
                user_table, item_table)


# speedup vs baseline: 2.0564x; 2.0564x over previous
import functools

import jax
import jax.numpy as jnp
from jax import lax
from jax.experimental import pallas as pl
from jax.experimental.pallas import tpu as pltpu
from jax.experimental.pallas import tpu_sc as plsc

_B = 16384
_D = 64
_NC = 2
_NS = 16
_NW = _NC * _NS
_BPW = _B // _NW
_WAVE = 16
_NWAVE = _BPW // _WAVE


def _body(users_hbm, items_hbm, ut3_hbm, it3_hbm, out_hbm,
          uidx_v, iidx_v, ublk_a, iblk_a, ublk_b, iblk_b, out_v, tile_v,
          sem_ua, sem_ia, sem_ub, sem_ib):
    cid = lax.axis_index("c")
    sid = lax.axis_index("s")
    wid = sid * _NC + cid
    base = wid * _BPW

    pltpu.sync_copy(users_hbm.at[pl.ds(base, _BPW)], uidx_v)
    pltpu.sync_copy(items_hbm.at[pl.ds(base, _BPW)], iidx_v)

    lanes = lax.iota(jnp.int32, 16)

    def fire(w, ublk, iblk, su, si):
        e0 = w * _WAVE
        ut_vec = lax.shift_right_logical(uidx_v[pl.ds(e0, 16)], 3)
        it_vec = lax.shift_right_logical(iidx_v[pl.ds(e0, 16)], 3)
        for j in range(_WAVE):
            pltpu.async_copy(ut3_hbm.at[ut_vec[j]], ublk.at[j], su)
            pltpu.async_copy(it3_hbm.at[it_vec[j]], iblk.at[j], si)

    def drain(ublk, iblk, su, si):
        pltpu.make_async_copy(ut3_hbm.at[pl.ds(0, _WAVE)], ublk, su).wait()
        pltpu.make_async_copy(it3_hbm.at[pl.ds(0, _WAVE)], iblk, si).wait()

    def compute(w, ublk, iblk):
        e0 = w * _WAVE
        us_vec = jnp.bitwise_and(uidx_v[pl.ds(e0, 16)], 7)
        is_vec = jnp.bitwise_and(iidx_v[pl.ds(e0, 16)], 7)
        for i in range(_WAVE):
            su = us_vec[i]
            si = is_vec[i]
            acc = ublk[i, su, pl.ds(0, 16)] * iblk[i, si, pl.ds(0, 16)]
            for c in range(1, 4):
                acc = acc + ublk[i, su, pl.ds(c * 16, 16)] * \
                    iblk[i, si, pl.ds(c * 16, 16)]
            plsc.store_scatter(tile_v, [lanes * 16 + i], acc)
        rowsum = tile_v[pl.ds(0, 16)]
        for r in range(1, 16):
            rowsum = rowsum + tile_v[pl.ds(r * 16, 16)]
        out_v[pl.ds(e0, 16)] = rowsum

    fire(0, ublk_a, iblk_a, sem_ua, sem_ia)

    def pair_body(k, carry):
        w0 = k * 2
        fire(w0 + 1, ublk_b, iblk_b, sem_ub, sem_ib)
        drain(ublk_a, iblk_a, sem_ua, sem_ia)
        compute(w0, ublk_a, iblk_a)

        @pl.when(k < _NWAVE // 2 - 1)
        def _():
            fire(w0 + 2, ublk_a, iblk_a, sem_ua, sem_ia)

        drain(ublk_b, iblk_b, sem_ub, sem_ib)
        compute(w0 + 1, ublk_b, iblk_b)
        return carry

    lax.fori_loop(0, _NWAVE // 2, pair_body, 0)

    pltpu.sync_copy(out_v, out_hbm.at[pl.ds(base, _BPW)])


def _tc_transpose(tT):
    n = tT.shape[1]
    blk = 2048
    grid = (n + blk - 1) // blk

    def body(i_ref, o_ref):
        eye = jnp.eye(_D, dtype=jnp.float32)
        o_ref[...] = jax.lax.dot_general(
            i_ref[...], eye,
            dimension_numbers=(((0,), (0,)), ((), ())),
            preferred_element_type=jnp.float32)

    return pl.pallas_call(
        body,
        grid=(grid,),
        in_specs=[pl.BlockSpec((_D, blk), lambda j: (0, j))],
        out_specs=pl.BlockSpec((blk, _D), lambda j: (j, 0)),
        out_shape=jax.ShapeDtypeStruct((n, _D), jnp.float32),
    )(tT)


@jax.jit
def _run(users, items, user_table, item_table):
    mesh = plsc.VectorSubcoreMesh(core_axis_name="c", subcore_axis_name="s")
    f = pl.kernel(
        _body,
        mesh=mesh,
        out_type=jax.ShapeDtypeStruct((_B,), jnp.float32),
        scratch_types=[
            pltpu.VMEM((_BPW,), jnp.int32),
            pltpu.VMEM((_BPW,), jnp.int32),
            pltpu.VMEM((_WAVE, 8, _D), jnp.float32),
            pltpu.VMEM((_WAVE, 8, _D), jnp.float32),
            pltpu.VMEM((_WAVE, 8, _D), jnp.float32),
            pltpu.VMEM((_WAVE, 8, _D), jnp.float32),
            pltpu.VMEM((_BPW,), jnp.float32),
            pltpu.VMEM((256,), jnp.float32),
            pltpu.SemaphoreType.DMA,
            pltpu.SemaphoreType.DMA,
            pltpu.SemaphoreType.DMA,
            pltpu.SemaphoreType.DMA,
        ],
        compiler_params=pltpu.CompilerParams(needs_layout_passes=False),
    )
    ut3 = _tc_transpose(user_table.T).reshape(125000, 8, _D)
    it3 = item_table.reshape(125000, 8, _D)
    return f(users, items, ut3, it3)


def kernel(users, items, user_table, item_table):
    return _run(users.astype(jnp.int32), items.astype(jnp.int32),
                user_table, item_table)


# speedup vs baseline: 2.0574x; 1.0005x over previous
import functools

import jax
import jax.numpy as jnp
from jax import lax
from jax.experimental import pallas as pl
from jax.experimental.pallas import tpu as pltpu
from jax.experimental.pallas import tpu_sc as plsc

_B = 16384
_D = 64
_NC = 2
_NS = 16
_NW = _NC * _NS
_BPW = _B // _NW
_WAVE = 16
_NWAVE = _BPW // _WAVE


def _body(users_hbm, items_hbm, ut3_hbm, it3_hbm, out_hbm,
          uidx_v, iidx_v, ublk_a, iblk_a, ublk_b, iblk_b, out_v, tile_v,
          sem_ua, sem_ia, sem_ub, sem_ib):
    cid = lax.axis_index("c")
    sid = lax.axis_index("s")
    wid = sid * _NC + cid
    base = wid * _BPW

    pltpu.sync_copy(users_hbm.at[pl.ds(base, _BPW)], uidx_v)
    pltpu.sync_copy(items_hbm.at[pl.ds(base, _BPW)], iidx_v)

    lanes = lax.iota(jnp.int32, 16)

    def fire(w, ublk, iblk, su, si):
        e0 = w * _WAVE
        ut_vec = lax.shift_right_logical(uidx_v[pl.ds(e0, 16)], 3)
        it_vec = lax.shift_right_logical(iidx_v[pl.ds(e0, 16)], 3)
        for j in range(_WAVE):
            pltpu.async_copy(ut3_hbm.at[ut_vec[j]], ublk.at[j], su)
            pltpu.async_copy(it3_hbm.at[it_vec[j]], iblk.at[j], si)

    def drain(ublk, iblk, su, si):
        pltpu.make_async_copy(ut3_hbm.at[pl.ds(0, _WAVE)], ublk, su).wait()
        pltpu.make_async_copy(it3_hbm.at[pl.ds(0, _WAVE)], iblk, si).wait()

    def compute(w, ublk, iblk):
        e0 = w * _WAVE
        us_vec = jnp.bitwise_and(uidx_v[pl.ds(e0, 16)], 7)
        is_vec = jnp.bitwise_and(iidx_v[pl.ds(e0, 16)], 7)
        for i in range(_WAVE):
            su = us_vec[i]
            si = is_vec[i]
            acc = ublk[i, su, pl.ds(0, 16)] * iblk[i, si, pl.ds(0, 16)]
            for c in range(1, 4):
                acc = acc + ublk[i, su, pl.ds(c * 16, 16)] * \
                    iblk[i, si, pl.ds(c * 16, 16)]
            plsc.store_scatter(tile_v, [lanes * 16 + i], acc)
        rowsum = tile_v[pl.ds(0, 16)]
        for r in range(1, 16):
            rowsum = rowsum + tile_v[pl.ds(r * 16, 16)]
        out_v[pl.ds(e0, 16)] = rowsum

    fire(0, ublk_a, iblk_a, sem_ua, sem_ia)

    def pair_body(k, carry):
        w0 = k * 2
        fire(w0 + 1, ublk_b, iblk_b, sem_ub, sem_ib)
        drain(ublk_a, iblk_a, sem_ua, sem_ia)
        compute(w0, ublk_a, iblk_a)

        @pl.when(k < _NWAVE // 2 - 1)
        def _():
            fire(w0 + 2, ublk_a, iblk_a, sem_ua, sem_ia)

        drain(ublk_b, iblk_b, sem_ub, sem_ib)
        compute(w0 + 1, ublk_b, iblk_b)
        return carry

    lax.fori_loop(0, _NWAVE // 2, pair_body, 0)

    pltpu.sync_copy(out_v, out_hbm.at[pl.ds(base, _BPW)])


def _tc_transpose(tT):
    n = tT.shape[1]
    blk = 2048
    grid = (n + blk - 1) // blk

    def body(i_ref, o_ref):
        eye = jnp.eye(_D, dtype=jnp.float32)
        o_ref[...] = jax.lax.dot_general(
            i_ref[...], eye,
            dimension_numbers=(((0,), (0,)), ((), ())),
            preferred_element_type=jnp.float32)

    return pl.pallas_call(
        body,
        grid=(grid,),
        in_specs=[pl.BlockSpec((_D, blk), lambda j: (0, j))],
        out_specs=pl.BlockSpec((blk, _D), lambda j: (j, 0)),
        out_shape=jax.ShapeDtypeStruct((n, _D), jnp.float32),
        compiler_params=pltpu.CompilerParams(skip_device_barrier=True),
    )(tT)


@jax.jit
def _run(users, items, user_table, item_table):
    mesh = plsc.VectorSubcoreMesh(core_axis_name="c", subcore_axis_name="s")
    f = pl.kernel(
        _body,
        mesh=mesh,
        out_type=jax.ShapeDtypeStruct((_B,), jnp.float32),
        scratch_types=[
            pltpu.VMEM((_BPW,), jnp.int32),
            pltpu.VMEM((_BPW,), jnp.int32),
            pltpu.VMEM((_WAVE, 8, _D), jnp.float32),
            pltpu.VMEM((_WAVE, 8, _D), jnp.float32),
            pltpu.VMEM((_WAVE, 8, _D), jnp.float32),
            pltpu.VMEM((_WAVE, 8, _D), jnp.float32),
            pltpu.VMEM((_BPW,), jnp.float32),
            pltpu.VMEM((256,), jnp.float32),
            pltpu.SemaphoreType.DMA,
            pltpu.SemaphoreType.DMA,
            pltpu.SemaphoreType.DMA,
            pltpu.SemaphoreType.DMA,
        ],
        compiler_params=pltpu.CompilerParams(
            needs_layout_passes=False, skip_device_barrier=True),
    )
    ut3 = _tc_transpose(user_table.T).reshape(125000, 8, _D)
    it3 = item_table.reshape(125000, 8, _D)
    return f(users, items, ut3, it3)


def kernel(users, items, user_table, item_table):
    return _run(users.astype(jnp.int32), items.astype(jnp.int32),
                user_table, item_table)


# speedup vs baseline: 2.6978x; 1.3113x over previous
import functools

import jax
import jax.numpy as jnp
from jax import lax
from jax.experimental import pallas as pl
from jax.experimental.pallas import tpu as pltpu
from jax.experimental.pallas import tpu_sc as plsc

_B = 16384
_D = 64
_NC = 2
_NS = 16
_NW = _NC * _NS
_BPW = _B // _NW
_WAVE = 16
_NWAVE = _BPW // _WAVE


def _body(users_hbm, items_hbm, ut3_hbm, it3_hbm, out_hbm,
          uidx_v, iidx_v, ublk_a, iblk_a, ublk_b, iblk_b, out_v, tile_v,
          sem_ua, sem_ia, sem_ub, sem_ib):
    cid = lax.axis_index("c")
    sid = lax.axis_index("s")
    wid = sid * _NC + cid
    base = wid * _BPW

    pltpu.sync_copy(users_hbm.at[pl.ds(base, _BPW)], uidx_v)
    pltpu.sync_copy(items_hbm.at[pl.ds(base, _BPW)], iidx_v)

    lanes = lax.iota(jnp.int32, 16)

    def fire(w, ublk, iblk, su, si):
        e0 = w * _WAVE
        ut_vec = lax.shift_right_logical(uidx_v[pl.ds(e0, 16)], 3)
        it_vec = lax.shift_right_logical(iidx_v[pl.ds(e0, 16)], 3)
        for j in range(_WAVE):
            pltpu.async_copy(ut3_hbm.at[ut_vec[j]], ublk.at[j], su)
            pltpu.async_copy(it3_hbm.at[it_vec[j]], iblk.at[j], si)

    def drain(ublk, iblk, su, si):
        pltpu.make_async_copy(ut3_hbm.at[pl.ds(0, _WAVE)], ublk, su).wait()
        pltpu.make_async_copy(it3_hbm.at[pl.ds(0, _WAVE)], iblk, si).wait()

    def compute(w, ublk, iblk):
        e0 = w * _WAVE
        us_vec = jnp.bitwise_and(uidx_v[pl.ds(e0, 16)], 7)
        is_vec = jnp.bitwise_and(iidx_v[pl.ds(e0, 16)], 7)
        for i in range(_WAVE):
            su = us_vec[i]
            si = is_vec[i]
            acc = ublk[i, su, pl.ds(0, 16)] * iblk[i, si, pl.ds(0, 16)]
            for c in range(1, 4):
                acc = acc + ublk[i, su, pl.ds(c * 16, 16)] * \
                    iblk[i, si, pl.ds(c * 16, 16)]
            plsc.store_scatter(tile_v, [lanes * 16 + i], acc)
        rowsum = tile_v[pl.ds(0, 16)]
        for r in range(1, 16):
            rowsum = rowsum + tile_v[pl.ds(r * 16, 16)]
        out_v[pl.ds(e0, 16)] = rowsum

    fire(0, ublk_a, iblk_a, sem_ua, sem_ia)

    def pair_body(k, carry):
        w0 = k * 2
        fire(w0 + 1, ublk_b, iblk_b, sem_ub, sem_ib)
        drain(ublk_a, iblk_a, sem_ua, sem_ia)
        compute(w0, ublk_a, iblk_a)

        @pl.when(k < _NWAVE // 2 - 1)
        def _():
            fire(w0 + 2, ublk_a, iblk_a, sem_ua, sem_ia)

        drain(ublk_b, iblk_b, sem_ub, sem_ib)
        compute(w0 + 1, ublk_b, iblk_b)
        return carry

    lax.fori_loop(0, _NWAVE // 2, pair_body, 0)

    pltpu.sync_copy(out_v, out_hbm.at[pl.ds(base, _BPW)])


def _tc_transpose(tT):
    n = tT.shape[1]
    blk = 2048
    grid = (n + blk - 1) // blk

    def body(i_ref, o_ref):
        eye = jnp.eye(_D, dtype=jnp.float32)
        o_ref[...] = jax.lax.dot_general(
            i_ref[...], eye,
            dimension_numbers=(((0,), (0,)), ((), ())),
            preferred_element_type=jnp.float32)

    return pl.pallas_call(
        body,
        grid=(grid,),
        in_specs=[pl.BlockSpec((_D, blk), lambda j: (0, j))],
        out_specs=pl.BlockSpec((blk, _D), lambda j: (j, 0)),
        out_shape=jax.ShapeDtypeStruct((n, _D), jnp.float32),
        compiler_params=pltpu.CompilerParams(skip_device_barrier=True),
    )(tT)


@jax.jit
def _run(users, items, user_table, item_table):
    mesh = plsc.VectorSubcoreMesh(core_axis_name="c", subcore_axis_name="s")
    f = pl.kernel(
        _body,
        mesh=mesh,
        out_type=jax.ShapeDtypeStruct((_B,), jnp.float32),
        scratch_types=[
            pltpu.VMEM((_BPW,), jnp.int32),
            pltpu.VMEM((_BPW,), jnp.int32),
            pltpu.VMEM((_WAVE, 8, _D), jnp.float32),
            pltpu.VMEM((_WAVE, 8, _D), jnp.float32),
            pltpu.VMEM((_WAVE, 8, _D), jnp.float32),
            pltpu.VMEM((_WAVE, 8, _D), jnp.float32),
            pltpu.VMEM((_BPW,), jnp.float32),
            pltpu.VMEM((256,), jnp.float32),
            pltpu.SemaphoreType.DMA,
            pltpu.SemaphoreType.DMA,
            pltpu.SemaphoreType.DMA,
            pltpu.SemaphoreType.DMA,
        ],
        compiler_params=pltpu.CompilerParams(
            needs_layout_passes=False, skip_device_barrier=True),
    )
    ut3 = user_table.reshape(125000, 8, _D)
    it3 = item_table.reshape(125000, 8, _D)
    return f(users, items, ut3, it3)


def kernel(users, items, user_table, item_table):
    return _run(users.astype(jnp.int32), items.astype(jnp.int32),
                user_table, item_table)
